# baseline (speedup 1.0000x reference)
import jax
import jax.numpy as jnp
from jax.experimental import pallas as pl

K = 8
EPS = 1e-5
N = 1024
B = 2

_DN = (((0,), (0,)), ((), ()))


def _dot_chunked(xb, chunk=256):
    d = xb.shape[0]
    if d <= chunk:
        return jax.lax.dot_general(xb, xb, _DN, preferred_element_type=jnp.float32)
    bounds = [(c0, min(c0 + chunk, d)) for c0 in range(0, d, chunk)]
    parts = [jax.lax.dot_general(xb[a:b], xb[a:b], _DN,
                                 preferred_element_type=jnp.float32)
             for a, b in bounds]
    if d % chunk:
        parts = [parts[-1]] + parts[:-1]
    acc = parts[0]
    for p in parts[1:]:
        acc = acc + p
    return acc


def _knn_kernel(x_ref, idx_ref):
    xb = x_ref[0]
    t = _dot_chunked(xb)
    inner = -2.0 * t
    xx = jnp.sum(xb * xb, axis=0)
    pd = (0.0 - xx)[None, :] - inner
    pd = pd - xx[:, None]
    cols = jax.lax.broadcasted_iota(jnp.int32, (N, N), 1)
    work = pd
    rows = []
    for _ in range(K):
        m = jnp.max(work, axis=1, keepdims=True)
        sel = jnp.where(work == m, cols, N)
        j = jnp.min(sel, axis=1, keepdims=True)
        rows.append(j)
        work = jnp.where(cols == j, -jnp.inf, work)
    idx_ref[0] = jnp.concatenate(rows, axis=1).astype(jnp.int32)


def _knn(x):
    d = x.shape[1]
    f = pl.pallas_call(
        _knn_kernel,
        out_shape=jax.ShapeDtypeStruct((B, N, K), jnp.int32),
        grid=(B,),
        in_specs=[pl.BlockSpec((1, d, N), lambda b: (b, 0, 0))],
        out_specs=pl.BlockSpec((1, N, K), lambda b: (b, 0, 0)),
    )
    return f(x)


def _get_graph_feature(x):
    b, d, n = x.shape
    idx = _knn(x)
    xt = jnp.transpose(x, (0, 2, 1))
    feature = jax.vmap(lambda t, i: t[i])(xt, idx)
    center = jnp.broadcast_to(xt[:, :, None, :], (b, n, K, d))
    feat = jnp.concatenate([feature - center, center], axis=3)
    return jnp.transpose(feat, (0, 3, 1, 2))


def _conv_bn(x, W, g, bt, relu=True):
    y = jnp.einsum('oc,bcnk->bonk', W, x)
    mean = jnp.mean(y, axis=(0, 2, 3), keepdims=True)
    var = jnp.var(y, axis=(0, 2, 3), keepdims=True)
    y = g.reshape(1, -1, 1, 1) * (y - mean) / jnp.sqrt(var + EPS) + bt.reshape(1, -1, 1, 1)
    if relu:
        y = jax.nn.relu(y)
    return y


def kernel(x, W1, g1, b1, W2, g2, b2, W3, g3, b3, W4, g4, b4, W5, g5, b5, W6, g6, b6):
    npoints = x.shape[2]
    h = _get_graph_feature(x)
    h = _conv_bn(h, W1, g1, b1).max(axis=-1)
    pointfeat = h
    h = _get_graph_feature(h)
    h = _conv_bn(h, W2, g2, b2).max(axis=-1)
    h = _get_graph_feature(h)
    h = _conv_bn(h, W3, g3, b3, relu=False).max(axis=-1)
    h = jnp.max(h, axis=2)
    h = jnp.broadcast_to(h[:, :, None], (h.shape[0], 1024, npoints))
    h = jnp.concatenate([h, pointfeat], axis=1)
    h = _get_graph_feature(h)
    h = _conv_bn(h, W4, g4, b4).max(axis=-1)
    h = _get_graph_feature(h)
    h = _conv_bn(h, W5, g5, b5).max(axis=-1)
    h = _get_graph_feature(h)
    h = _conv_bn(h, W6, g6, b6).max(axis=-1)
    return h


# speedup vs baseline: 1.3903x; 1.3903x over previous
import functools

import jax
import jax.numpy as jnp
from jax.experimental import pallas as pl
from jax.experimental.pallas import tpu as pltpu
from jax.experimental.pallas import tpu_sc as plsc

K = 8
EPS = 1e-5
N = 1024
B = 2

_DN = (((0,), (0,)), ((), ()))


def _dot_chunked(xb, chunk=256):
    d = xb.shape[0]
    if d <= chunk:
        return jax.lax.dot_general(xb, xb, _DN, preferred_element_type=jnp.float32)
    bounds = [(c0, min(c0 + chunk, d)) for c0 in range(0, d, chunk)]
    parts = [jax.lax.dot_general(xb[a:b], xb[a:b], _DN,
                                 preferred_element_type=jnp.float32)
             for a, b in bounds]
    if d % chunk:
        parts = [parts[-1]] + parts[:-1]
    acc = parts[0]
    for p in parts[1:]:
        acc = acc + p
    return acc


def _knn_kernel(x_ref, idx_ref):
    xb = x_ref[0]
    t = _dot_chunked(xb)
    inner = -2.0 * t
    xx = jnp.sum(xb * xb, axis=0)
    pd = (0.0 - xx)[None, :] - inner
    pd = pd - xx[:, None]
    cols = jax.lax.broadcasted_iota(jnp.int32, (N, N), 1)
    work = pd
    rows = []
    for _ in range(K):
        m = jnp.max(work, axis=1, keepdims=True)
        sel = jnp.where(work == m, cols, N)
        j = jnp.min(sel, axis=1, keepdims=True)
        rows.append(j)
        work = jnp.where(cols == j, -jnp.inf, work)
    idx_ref[0] = jnp.concatenate(rows, axis=1).astype(jnp.int32)


def _knn(x):
    d = x.shape[1]
    f = pl.pallas_call(
        _knn_kernel,
        out_shape=jax.ShapeDtypeStruct((B, N, K), jnp.int32),
        grid=(B,),
        in_specs=[pl.BlockSpec((1, d, N), lambda b: (b, 0, 0))],
        out_specs=pl.BlockSpec((1, N, K), lambda b: (b, 0, 0)),
    )
    return f(x)


def _sc_gather(table, gidx, d):
    tot = gidx.shape[0]
    nw = 32
    per_w = tot // nw
    ch = 64
    mesh = plsc.VectorSubcoreMesh(core_axis_name="c", subcore_axis_name="s")

    @functools.partial(
        pl.kernel, mesh=mesh,
        out_type=jax.ShapeDtypeStruct((tot, d), jnp.float32),
        scratch_types=[pltpu.VMEM((ch,), jnp.int32),
                       pltpu.VMEM((ch, d), jnp.float32),
                       pltpu.SemaphoreType.DMA],
    )
    def k(table_hbm, idx_hbm, out_hbm, idx_v, rows_v, sem):
        wid = jax.lax.axis_index("s") * 2 + jax.lax.axis_index("c")
        base = wid * per_w
        for i in range(per_w // ch):
            off = base + i * ch
            pltpu.sync_copy(idx_hbm.at[pl.ds(off, ch)], idx_v)
            pltpu.async_copy(table_hbm.at[idx_v], rows_v, sem).wait()
            pltpu.sync_copy(rows_v, out_hbm.at[pl.ds(off, ch)])

    return k(table, gidx)


def _get_graph_feature(x):
    b, d, n = x.shape
    idx = _knn(x)
    xt = jnp.transpose(x, (0, 2, 1))
    if d % 128 == 0:
        table = xt.reshape(b * n, d)
        gidx = (idx + (jnp.arange(b, dtype=jnp.int32) * n)[:, None, None]).reshape(-1)
        feature = _sc_gather(table, gidx, d).reshape(b, n, K, d)
    else:
        feature = jax.vmap(lambda t, i: t[i])(xt, idx)
    center = jnp.broadcast_to(xt[:, :, None, :], (b, n, K, d))
    feat = jnp.concatenate([feature - center, center], axis=3)
    return jnp.transpose(feat, (0, 3, 1, 2))


def _conv_bn(x, W, g, bt, relu=True):
    y = jnp.einsum('oc,bcnk->bonk', W, x)
    mean = jnp.mean(y, axis=(0, 2, 3), keepdims=True)
    var = jnp.var(y, axis=(0, 2, 3), keepdims=True)
    y = g.reshape(1, -1, 1, 1) * (y - mean) / jnp.sqrt(var + EPS) + bt.reshape(1, -1, 1, 1)
    if relu:
        y = jax.nn.relu(y)
    return y


def kernel(x, W1, g1, b1, W2, g2, b2, W3, g3, b3, W4, g4, b4, W5, g5, b5, W6, g6, b6):
    npoints = x.shape[2]
    h = _get_graph_feature(x)
    h = _conv_bn(h, W1, g1, b1).max(axis=-1)
    pointfeat = h
    h = _get_graph_feature(h)
    h = _conv_bn(h, W2, g2, b2).max(axis=-1)
    h = _get_graph_feature(h)
    h = _conv_bn(h, W3, g3, b3, relu=False).max(axis=-1)
    h = jnp.max(h, axis=2)
    h = jnp.broadcast_to(h[:, :, None], (h.shape[0], 1024, npoints))
    h = jnp.concatenate([h, pointfeat], axis=1)
    h = _get_graph_feature(h)
    h = _conv_bn(h, W4, g4, b4).max(axis=-1)
    h = _get_graph_feature(h)
    h = _conv_bn(h, W5, g5, b5).max(axis=-1)
    h = _get_graph_feature(h)
    h = _conv_bn(h, W6, g6, b6).max(axis=-1)
    return h


# speedup vs baseline: 2.7133x; 1.9516x over previous
import functools

import jax
import jax.numpy as jnp
from jax.experimental import pallas as pl
from jax.experimental.pallas import tpu as pltpu
from jax.experimental.pallas import tpu_sc as plsc

K = 8
EPS = 1e-5
N = 1024
B = 2

_DN = (((0,), (0,)), ((), ()))


def _dot_chunked(xb, chunk=256):
    d = xb.shape[0]
    if d <= chunk:
        return jax.lax.dot_general(xb, xb, _DN, preferred_element_type=jnp.float32)
    bounds = [(c0, min(c0 + chunk, d)) for c0 in range(0, d, chunk)]
    parts = [jax.lax.dot_general(xb[a:b], xb[a:b], _DN,
                                 preferred_element_type=jnp.float32)
             for a, b in bounds]
    if d % chunk:
        parts = [parts[-1]] + parts[:-1]
    acc = parts[0]
    for p in parts[1:]:
        acc = acc + p
    return acc


def _knn_kernel(x_ref, idx_ref):
    xb = x_ref[0]
    t = _dot_chunked(xb)
    inner = -2.0 * t
    xx = jnp.sum(xb * xb, axis=0)
    pd = (0.0 - xx)[None, :] - inner
    pd = pd - xx[:, None]
    cols = jax.lax.broadcasted_iota(jnp.int32, (N, N), 1)
    work = pd
    rows = []
    for _ in range(K):
        m = jnp.max(work, axis=1, keepdims=True)
        sel = jnp.where(work == m, cols, N)
        j = jnp.min(sel, axis=1, keepdims=True)
        rows.append(j)
        work = jnp.where(cols == j, -jnp.inf, work)
    idx_ref[0] = jnp.concatenate(rows, axis=1).astype(jnp.int32)


def _knn(x):
    d = x.shape[1]
    f = pl.pallas_call(
        _knn_kernel,
        out_shape=jax.ShapeDtypeStruct((B, N, K), jnp.int32),
        grid=(B,),
        in_specs=[pl.BlockSpec((1, d, N), lambda b: (b, 0, 0))],
        out_specs=pl.BlockSpec((1, N, K), lambda b: (b, 0, 0)),
    )
    return f(x)


def _sc_gather(table, gidx, d):
    tot = gidx.shape[0]
    nw = 32
    per_w = tot // nw
    ch = 64
    mesh = plsc.VectorSubcoreMesh(core_axis_name="c", subcore_axis_name="s")

    @functools.partial(
        pl.kernel, mesh=mesh,
        out_type=jax.ShapeDtypeStruct((tot, d), jnp.float32),
        scratch_types=[pltpu.VMEM((ch,), jnp.int32),
                       pltpu.VMEM((ch, d), jnp.float32),
                       pltpu.SemaphoreType.DMA],
    )
    def k(table_hbm, idx_hbm, out_hbm, idx_v, rows_v, sem):
        wid = jax.lax.axis_index("s") * 2 + jax.lax.axis_index("c")
        base = wid * per_w
        for i in range(per_w // ch):
            off = base + i * ch
            pltpu.sync_copy(idx_hbm.at[pl.ds(off, ch)], idx_v)
            pltpu.async_copy(table_hbm.at[idx_v], rows_v, sem).wait()
            pltpu.sync_copy(rows_v, out_hbm.at[pl.ds(off, ch)])

    return k(table, gidx)


def _get_graph_feature(x):
    b, d, n = x.shape
    idx = _knn(x)
    xt = jnp.transpose(x, (0, 2, 1))
    dp = -(-d // 128) * 128
    table = xt.reshape(b * n, d)
    if dp != d:
        table = jnp.pad(table, ((0, 0), (0, dp - d)))
    gidx = (idx + (jnp.arange(b, dtype=jnp.int32) * n)[:, None, None]).reshape(-1)
    feature = _sc_gather(table, gidx, dp)[:, :d].reshape(b, n, K, d)
    center = jnp.broadcast_to(xt[:, :, None, :], (b, n, K, d))
    feat = jnp.concatenate([feature - center, center], axis=3)
    return jnp.transpose(feat, (0, 3, 1, 2))


def _conv_bn(x, W, g, bt, relu=True):
    y = jnp.einsum('oc,bcnk->bonk', W, x)
    mean = jnp.mean(y, axis=(0, 2, 3), keepdims=True)
    var = jnp.var(y, axis=(0, 2, 3), keepdims=True)
    y = g.reshape(1, -1, 1, 1) * (y - mean) / jnp.sqrt(var + EPS) + bt.reshape(1, -1, 1, 1)
    if relu:
        y = jax.nn.relu(y)
    return y


def kernel(x, W1, g1, b1, W2, g2, b2, W3, g3, b3, W4, g4, b4, W5, g5, b5, W6, g6, b6):
    npoints = x.shape[2]
    h = _get_graph_feature(x)
    h = _conv_bn(h, W1, g1, b1).max(axis=-1)
    pointfeat = h
    h = _get_graph_feature(h)
    h = _conv_bn(h, W2, g2, b2).max(axis=-1)
    h = _get_graph_feature(h)
    h = _conv_bn(h, W3, g3, b3, relu=False).max(axis=-1)
    h = jnp.max(h, axis=2)
    h = jnp.broadcast_to(h[:, :, None], (h.shape[0], 1024, npoints))
    h = jnp.concatenate([h, pointfeat], axis=1)
    h = _get_graph_feature(h)
    h = _conv_bn(h, W4, g4, b4).max(axis=-1)
    h = _get_graph_feature(h)
    h = _conv_bn(h, W5, g5, b5).max(axis=-1)
    h = _get_graph_feature(h)
    h = _conv_bn(h, W6, g6, b6).max(axis=-1)
    return h
